# baseline (speedup 1.0000x reference)
import functools

import jax
import jax.numpy as jnp
from jax import lax
from jax.experimental import pallas as pl
from jax.experimental.pallas import tpu as pltpu
from jax.experimental.pallas import tpu_sc as plsc

NUM_FIELDS = 26
VOCAB = 100000
DIM = 16
BATCH = 16384

NC, NS, L = 2, 16, 16
NW = NC * NS
NROWS = NUM_FIELDS * DIM

SPLITS = [(0, 224, 7), (224, 192, 6)]

_mesh = plsc.VectorSubcoreMesh(core_axis_name="c", subcore_axis_name="s")


def _make_embed(r0, rs, pw):
    @functools.partial(
        pl.kernel,
        mesh=_mesh,
        compiler_params=pltpu.CompilerParams(use_tc_tiling_on_sc=False),
        out_type=jax.ShapeDtypeStruct((rs, BATCH), jnp.float32),
        scratch_types=[
            pltpu.VMEM((2, BATCH), jnp.int32),
            pltpu.VMEM((BATCH,), jnp.float32),
            pltpu.VMEM((BATCH,), jnp.float32),
            pltpu.SemaphoreType.DMA,
            pltpu.SemaphoreType.DMA,
            pltpu.SemaphoreType.DMA,
            pltpu.SemaphoreType.DMA,
            pltpu.SemaphoreType.DMA,
        ],
    )
    def _embed(feat_hbm, tbl_hbm, out_hbm, idx_v, ga, gb,
               isem, gsem_a, gsem_b, ssem_a, ssem_b):
        wid = lax.axis_index("s") * NC + lax.axis_index("c")
        p0 = wid * pw
        f0 = lax.shift_right_logical(r0 + p0, 4)
        f1 = lax.min(f0 + 1, NUM_FIELDS - 1)

        pltpu.async_copy(feat_hbm.at[f0], idx_v.at[0], isem)
        pltpu.async_copy(feat_hbm.at[f1], idx_v.at[1], isem)

        def idx_of(p):
            return idx_v.at[lax.shift_right_logical(r0 + p, 4) - f0]

        def fire_gather(p, buf, sem):
            pltpu.async_copy(tbl_hbm.at[p].at[idx_of(p)], buf, sem)

        def wait_gather(buf, sem):
            pltpu.make_async_copy(
                tbl_hbm.at[0].at[idx_v.at[0]], buf, sem).wait()

        def fire_store(p, buf, sem):
            pltpu.async_copy(buf, out_hbm.at[p], sem)

        def wait_store(buf, sem):
            pltpu.make_async_copy(buf, out_hbm.at[0], sem).wait()

        pltpu.make_async_copy(feat_hbm.at[0], idx_v.at[0], isem).wait()
        pltpu.make_async_copy(feat_hbm.at[0], idx_v.at[1], isem).wait()
        fire_gather(p0, ga, gsem_a)

        def body(i, _):
            pa = p0 + 2 * i
            pb = pa + 1
            pl.when(i > 0)(lambda: wait_store(gb, ssem_b))
            pl.when(2 * i + 1 < pw)(lambda: fire_gather(pb, gb, gsem_b))
            wait_gather(ga, gsem_a)
            fire_store(pa, ga, ssem_a)

            @pl.when(2 * i + 1 < pw)
            def _():
                wait_gather(gb, gsem_b)
                fire_store(pb, gb, ssem_b)

            wait_store(ga, ssem_a)

            @pl.when(2 * i + 2 < pw)
            def _():
                fire_gather(pa + 2, ga, gsem_a)

            return 0

        lax.fori_loop(0, (pw + 1) // 2, body, 0)
        if pw % 2 == 0:
            wait_store(gb, ssem_b)

    return _embed


def kernel(features, tables):
    feat_t = features.T.astype(jnp.int32)
    tbl = tables.transpose(0, 2, 1).reshape(NROWS, VOCAB)
    outs = [
        _make_embed(r0, rs, pw)(feat_t, tbl[r0:r0 + rs])
        for (r0, rs, pw) in SPLITS
    ]
    out_t = jnp.concatenate(outs, axis=0)
    return out_t.T


# speedup vs baseline: 1.0782x; 1.0782x over previous
import functools

import jax
import jax.numpy as jnp
from jax import lax
from jax.experimental import pallas as pl
from jax.experimental.pallas import tpu as pltpu
from jax.experimental.pallas import tpu_sc as plsc

NUM_FIELDS = 26
VOCAB = 100000
DIM = 16
BATCH = 16384

NC, NS, L = 2, 16, 16
NW = NC * NS
NROWS = NUM_FIELDS * DIM
PAIRS_W = NROWS // NW

_mesh = plsc.VectorSubcoreMesh(core_axis_name="c", subcore_axis_name="s")


@functools.partial(
    pl.kernel,
    mesh=_mesh,
    compiler_params=pltpu.CompilerParams(use_tc_tiling_on_sc=False),
    out_type=jax.ShapeDtypeStruct((NROWS, BATCH), jnp.float32),
    scratch_types=[
        pltpu.VMEM((2, BATCH), jnp.int32),
        pltpu.VMEM((BATCH,), jnp.float32),
        pltpu.VMEM((BATCH,), jnp.float32),
        pltpu.SemaphoreType.DMA,
        pltpu.SemaphoreType.DMA,
        pltpu.SemaphoreType.DMA,
        pltpu.SemaphoreType.DMA,
        pltpu.SemaphoreType.DMA,
    ],
)
def _embed(feat_hbm, table_hbm, out_hbm, idx_v, ga, gb,
           isem, gsem_a, gsem_b, ssem_a, ssem_b):
    wid = lax.axis_index("s") * NC + lax.axis_index("c")
    p0 = wid * PAIRS_W
    f0 = lax.shift_right_logical(p0, 4)
    f1 = lax.min(f0 + 1, NUM_FIELDS - 1)

    pltpu.async_copy(feat_hbm.at[f0], idx_v.at[0], isem)
    pltpu.async_copy(feat_hbm.at[f1], idx_v.at[1], isem)

    def idx_of(p):
        return idx_v.at[lax.shift_right_logical(p, 4) - f0]

    def fire_gather(p, buf, sem):
        pltpu.async_copy(table_hbm.at[p].at[idx_of(p)], buf, sem)

    def wait_gather(buf, sem):
        pltpu.make_async_copy(table_hbm.at[0].at[idx_v.at[0]], buf, sem).wait()

    def fire_store(p, buf, sem):
        pltpu.async_copy(buf, out_hbm.at[p], sem)

    def wait_store(buf, sem):
        pltpu.make_async_copy(buf, out_hbm.at[0], sem).wait()

    pltpu.make_async_copy(feat_hbm.at[0], idx_v.at[0], isem).wait()
    pltpu.make_async_copy(feat_hbm.at[0], idx_v.at[1], isem).wait()
    fire_gather(p0, ga, gsem_a)

    def body(i, _):
        pa = p0 + 2 * i
        pb = pa + 1
        pl.when(i > 0)(lambda: wait_store(gb, ssem_b))
        pl.when(2 * i + 1 < PAIRS_W)(lambda: fire_gather(pb, gb, gsem_b))
        wait_gather(ga, gsem_a)
        fire_store(pa, ga, ssem_a)

        @pl.when(2 * i + 1 < PAIRS_W)
        def _():
            wait_gather(gb, gsem_b)
            fire_store(pb, gb, ssem_b)

        wait_store(ga, ssem_a)

        @pl.when(2 * i + 2 < PAIRS_W)
        def _():
            fire_gather(pa + 2, ga, gsem_a)

        return 0

    lax.fori_loop(0, (PAIRS_W + 1) // 2, body, 0)
    if PAIRS_W % 2 == 0:
        wait_store(gb, ssem_b)


def kernel(features, tables):
    feat_t = features.T.astype(jnp.int32)
    tbl = tables.transpose(0, 2, 1).reshape(NROWS, VOCAB)
    out_t = _embed(feat_t, tbl)
    return out_t.T
